# baseline (speedup 1.0000x reference)
import functools

import jax
import jax.numpy as jnp
from jax import lax
from jax.experimental import pallas as pl
from jax.experimental.pallas import tpu as pltpu
from jax.experimental.pallas import tpu_sc as plsc

N_NODES = 10000
N_EDGES = 320000
D_FEAT = 128

NC = 2
NS = 16
NW = NC * NS
EPW = N_EDGES // NW
NPAD = 10240
ZSL = NPAD // NS

_EPS = 1e-12


def _alpha_body(h_ref, t_ref, o_ref):
    h = h_ref[...]
    t = t_ref[...]
    ones = jnp.ones((D_FEAT, 1), jnp.float32)
    dot = jax.lax.dot(h * t, ones).reshape(o_ref.shape)
    hs = jax.lax.dot(h * h, ones).reshape(o_ref.shape)
    ts = jax.lax.dot(t * t, ones).reshape(o_ref.shape)
    denom_sq = jnp.maximum(hs, _EPS * _EPS) * jnp.maximum(ts, _EPS * _EPS)
    o_ref[...] = dot * (0.5 * lax.rsqrt(denom_sq)) + 0.5


def _alpha_tc(head_embeds, tail_embeds):
    blk = 8000
    rows, cols = 8, blk // 8
    grid = (N_EDGES // blk,)
    return pl.pallas_call(
        _alpha_body,
        grid=grid,
        in_specs=[
            pl.BlockSpec((blk, D_FEAT), lambda i: (i, 0)),
            pl.BlockSpec((blk, D_FEAT), lambda i: (i, 0)),
        ],
        out_specs=pl.BlockSpec((rows, cols), lambda i: (i, 0)),
        out_shape=jax.ShapeDtypeStruct(
            (N_EDGES // cols, cols), jnp.float32),
    )(head_embeds, tail_embeds)


def _segment_sum_body(idx_hbm, alpha_hbm, out_hbm, idx_v, alpha_v, zbuf, d_sh,
                      sem_i, sem_a):
    c = lax.axis_index("c")
    s = lax.axis_index("s")
    wid = s * NC + c
    base = wid * EPW

    cp_i = pltpu.async_copy(idx_hbm.at[pl.ds(base, EPW)], idx_v, sem_i)
    cp_a = pltpu.async_copy(alpha_hbm.at[pl.ds(base, EPW)], alpha_v, sem_a)

    @plsc.parallel_loop(0, ZSL, step=16, unroll=4)
    def zero_body(i):
        zbuf[pl.ds(i, 16)] = jnp.zeros((16,), jnp.float32)

    pltpu.sync_copy(zbuf, d_sh.at[pl.ds(s * ZSL, ZSL)])
    cp_i.wait()
    cp_a.wait()
    plsc.subcore_barrier()
    pltpu.sync_copy(alpha_v, d_sh.at[idx_v], add=True)
    plsc.subcore_barrier()

    @pl.when(s == 0)
    def _():
        pltpu.sync_copy(d_sh, out_hbm.at[pl.ds(c * NPAD, NPAD)])


def _finalize_body(dpart_hbm, idx_hbm, alpha_hbm, out_hbm, a_v, b_v, d_full,
                   idx_v, alpha_v, out_v, d_sh, sem_i, sem_a):
    c = lax.axis_index("c")
    s = lax.axis_index("s")
    wid = s * NC + c
    base = wid * EPW

    cp_i = pltpu.async_copy(idx_hbm.at[pl.ds(base, EPW)], idx_v, sem_i)
    cp_a = pltpu.async_copy(alpha_hbm.at[pl.ds(base, EPW)], alpha_v, sem_a)

    pltpu.sync_copy(dpart_hbm.at[pl.ds(s * ZSL, ZSL)], a_v)
    pltpu.sync_copy(dpart_hbm.at[pl.ds(NPAD + s * ZSL, ZSL)], b_v)

    @plsc.parallel_loop(0, ZSL, step=16, unroll=4)
    def inv_body(i):
        sl = pl.ds(i, 16)
        dsum = a_v[sl] + b_v[sl]
        a_v[sl] = jnp.where(dsum != 0.0, 1.0 / dsum, 0.0)

    pltpu.sync_copy(a_v, d_sh.at[pl.ds(s * ZSL, ZSL)])
    plsc.subcore_barrier()
    pltpu.sync_copy(d_sh, d_full)
    cp_i.wait()
    cp_a.wait()

    @plsc.parallel_loop(0, EPW, step=16, unroll=4)
    def gather_body(i):
        sl = pl.ds(i, 16)
        out_v[sl] = plsc.load_gather(d_full, [idx_v[sl]]) * alpha_v[sl]

    pltpu.sync_copy(out_v, out_hbm.at[pl.ds(base, EPW)])


@functools.lru_cache(maxsize=1)
def _sc_kernels():
    mesh = plsc.VectorSubcoreMesh(core_axis_name="c", subcore_axis_name="s")
    params = pltpu.CompilerParams(needs_layout_passes=False)
    segment_sum = pl.kernel(
        _segment_sum_body,
        out_type=jax.ShapeDtypeStruct((NC * NPAD,), jnp.float32),
        mesh=mesh,
        compiler_params=params,
        scratch_types=[
            pltpu.VMEM((EPW,), jnp.int32),
            pltpu.VMEM((EPW,), jnp.float32),
            pltpu.VMEM((ZSL,), jnp.float32),
            pltpu.VMEM_SHARED((NPAD,), jnp.float32),
            pltpu.SemaphoreType.DMA,
            pltpu.SemaphoreType.DMA,
        ],
    )
    finalize = pl.kernel(
        _finalize_body,
        out_type=jax.ShapeDtypeStruct((N_EDGES,), jnp.float32),
        mesh=mesh,
        compiler_params=params,
        scratch_types=[
            pltpu.VMEM((ZSL,), jnp.float32),
            pltpu.VMEM((ZSL,), jnp.float32),
            pltpu.VMEM((NPAD,), jnp.float32),
            pltpu.VMEM((EPW,), jnp.int32),
            pltpu.VMEM((EPW,), jnp.float32),
            pltpu.VMEM((EPW,), jnp.float32),
            pltpu.VMEM_SHARED((NPAD,), jnp.float32),
            pltpu.SemaphoreType.DMA,
            pltpu.SemaphoreType.DMA,
        ],
    )
    return segment_sum, finalize


def kernel(head_embeds, tail_embeds, head_list, tail_list):
    segment_sum_sc, finalize_sc = _sc_kernels()
    alpha = _alpha_tc(head_embeds, tail_embeds).reshape(N_EDGES)
    d_partial = segment_sum_sc(head_list, alpha)
    g_values = finalize_sc(d_partial, head_list, alpha)
    g_indices = jnp.stack([head_list, tail_list], axis=0)
    return (g_indices, g_values)


# speedup vs baseline: 1.1150x; 1.1150x over previous
import functools

import jax
import jax.numpy as jnp
from jax import lax
from jax.experimental import pallas as pl
from jax.experimental.pallas import tpu as pltpu
from jax.experimental.pallas import tpu_sc as plsc

N_NODES = 10000
N_EDGES = 320000
D_FEAT = 128

NC = 2
NS = 16
NW = NC * NS
EPW = N_EDGES // NW
NPAD = 10240
ZSL = NPAD // NS

_EPS = 1e-12


def _alpha_body(h_ref, t_ref, o_ref):
    h = h_ref[...]
    t = t_ref[...]
    ones = jnp.ones((D_FEAT, 1), jnp.float32)
    dot = jax.lax.dot(h * t, ones).reshape(o_ref.shape)
    hs = jax.lax.dot(h * h, ones).reshape(o_ref.shape)
    ts = jax.lax.dot(t * t, ones).reshape(o_ref.shape)
    denom_sq = jnp.maximum(hs, _EPS * _EPS) * jnp.maximum(ts, _EPS * _EPS)
    o_ref[...] = dot * (0.5 * lax.rsqrt(denom_sq)) + 0.5


def _alpha_tc(head_embeds, tail_embeds):
    blk = 16000
    rows, cols = 8, blk // 8
    grid = (N_EDGES // blk,)
    return pl.pallas_call(
        _alpha_body,
        grid=grid,
        in_specs=[
            pl.BlockSpec((blk, D_FEAT), lambda i: (i, 0)),
            pl.BlockSpec((blk, D_FEAT), lambda i: (i, 0)),
        ],
        out_specs=pl.BlockSpec((rows, cols), lambda i: (i, 0)),
        out_shape=jax.ShapeDtypeStruct(
            (N_EDGES // cols, cols), jnp.float32),
    )(head_embeds, tail_embeds)


EPT = N_EDGES // NS


def _fused_sc_body(idx_hbm, alpha_hbm, out_hbm, idx_v, alpha_v, a_v, d_full,
                   out_v, d_sh, sem_i, sem_a):
    c = lax.axis_index("c")
    s = lax.axis_index("s")
    sbase = s * EPT
    gbase = sbase + c * EPW

    cp_i = pltpu.async_copy(idx_hbm.at[pl.ds(sbase, EPT)], idx_v, sem_i)
    cp_a = pltpu.async_copy(alpha_hbm.at[pl.ds(sbase, EPT)], alpha_v, sem_a)

    @plsc.parallel_loop(0, ZSL, step=16, unroll=4)
    def zero_body(i):
        a_v[pl.ds(i, 16)] = jnp.zeros((16,), jnp.float32)

    pltpu.sync_copy(a_v, d_sh.at[pl.ds(s * ZSL, ZSL)])
    cp_i.wait()
    cp_a.wait()
    plsc.subcore_barrier()
    pltpu.sync_copy(alpha_v, d_sh.at[idx_v], add=True)
    plsc.subcore_barrier()

    pltpu.sync_copy(d_sh.at[pl.ds(s * ZSL, ZSL)], a_v)

    @plsc.parallel_loop(0, ZSL, step=16, unroll=4)
    def inv_body(i):
        sl = pl.ds(i, 16)
        dsum = a_v[sl]
        a_v[sl] = jnp.where(dsum != 0.0, 1.0 / dsum, 0.0)

    pltpu.sync_copy(a_v, d_sh.at[pl.ds(s * ZSL, ZSL)])
    plsc.subcore_barrier()
    pltpu.sync_copy(d_sh, d_full)

    @plsc.parallel_loop(0, EPW, step=16, unroll=4)
    def gather_body(i):
        sl = pl.ds(c * EPW + i, 16)
        out_v[pl.ds(i, 16)] = (
            plsc.load_gather(d_full, [idx_v[sl]]) * alpha_v[sl])

    pltpu.sync_copy(out_v, out_hbm.at[pl.ds(gbase, EPW)])


@functools.lru_cache(maxsize=1)
def _sc_kernels():
    mesh = plsc.VectorSubcoreMesh(core_axis_name="c", subcore_axis_name="s")
    params = pltpu.CompilerParams(needs_layout_passes=False)
    fused = pl.kernel(
        _fused_sc_body,
        out_type=jax.ShapeDtypeStruct((N_EDGES,), jnp.float32),
        mesh=mesh,
        compiler_params=params,
        scratch_types=[
            pltpu.VMEM((EPT,), jnp.int32),
            pltpu.VMEM((EPT,), jnp.float32),
            pltpu.VMEM((ZSL,), jnp.float32),
            pltpu.VMEM((NPAD,), jnp.float32),
            pltpu.VMEM((EPW,), jnp.float32),
            pltpu.VMEM_SHARED((NPAD,), jnp.float32),
            pltpu.SemaphoreType.DMA,
            pltpu.SemaphoreType.DMA,
        ],
    )
    return fused


def kernel(head_embeds, tail_embeds, head_list, tail_list):
    fused_sc = _sc_kernels()
    alpha = _alpha_tc(head_embeds, tail_embeds).reshape(N_EDGES)
    g_values = fused_sc(head_list, alpha)
    g_indices = jnp.stack([head_list, tail_list], axis=0)
    return (g_indices, g_values)
